# baseline (speedup 1.0000x reference)
import functools

import jax
import jax.numpy as jnp
import numpy as np
from jax import lax
from jax.experimental import pallas as pl
from jax.experimental.pallas import tpu as pltpu
from jax.experimental.pallas import tpu_sc as plsc

N = 10000
E = 320000
D = 128
DE = 4
EDGE_DIM = 2
INJ_DIM = 2
NMASK = max(1, int(N * 0.15))
EMASK = max(1, int(E * 0.15))

NC = 2
NS = 16
LANES = 128
K1 = 1
CH1 = K1 * LANES
ITERS1 = 160
E_PAD = NS * ITERS1 * CH1
NP = N + 16

K2 = 2
CH2 = K2 * LANES
ITERS2 = 6
PAIR_PAD = NC * NS * ITERS2 * CH2

NODE_PAD = 1536
DEP = 16

_colmask_n = (np.arange(D) < INJ_DIM).astype(np.float32)[None, :]
_wmask_node = np.zeros((NODE_PAD, 1), np.float32)
_wmask_node[:NMASK] = 1.0
_wmask_edge = np.zeros((PAIR_PAD, 1), np.float32)
_wmask_edge[:EMASK] = 1.0


def _tc_h_body(x_ref, nm_ref, tok_ref, cm_ref, w_ref, b_ref, out_ref):
    x = x_ref[...]
    nm = nm_ref[...]
    xm = x * (1.0 - nm * cm_ref[...]) + nm * tok_ref[...]
    w = w_ref[...]
    b = b_ref[...]
    out_ref[0] = jax.nn.relu(jnp.dot(xm, w, preferred_element_type=jnp.float32) + b)
    out_ref[1] = jax.nn.relu(jnp.dot(x, w, preferred_element_type=jnp.float32) + b)


def _make_h(x, nmask_row, tokrow, colmask, W_in, b_in):
    blk = 1000
    return pl.pallas_call(
        _tc_h_body,
        grid=(N // blk,),
        in_specs=[
            pl.BlockSpec((blk, D), lambda i: (i, 0)),
            pl.BlockSpec((blk, 1), lambda i: (i, 0)),
            pl.BlockSpec((1, D), lambda i: (0, 0)),
            pl.BlockSpec((1, D), lambda i: (0, 0)),
            pl.BlockSpec((D, D), lambda i: (0, 0)),
            pl.BlockSpec((1, D), lambda i: (0, 0)),
        ],
        out_specs=pl.BlockSpec((2, blk, D), lambda i: (0, i, 0)),
        out_shape=jax.ShapeDtypeStruct((2, N, D), jnp.float32),
    )(x, nmask_row, tokrow, colmask, W_in, b_in)


def _tc_ea_body(ea_ref, sel_ref, tok_ref, out_ref):
    ea = ea_ref[...]
    sel = sel_ref[...]
    out_ref[0] = ea
    out_ref[1] = ea * (1.0 - sel) + sel * tok_ref[...]


def _make_ea(ea_r, esel_r, etok128):
    rows = E_PAD * DEP // 128
    blk = 2048
    return pl.pallas_call(
        _tc_ea_body,
        grid=(rows // blk,),
        in_specs=[
            pl.BlockSpec((blk, 128), lambda i: (i, 0)),
            pl.BlockSpec((blk, 128), lambda i: (i, 0)),
            pl.BlockSpec((1, 128), lambda i: (0, 0)),
        ],
        out_specs=pl.BlockSpec((2, blk, 128), lambda i: (0, i, 0)),
        out_shape=jax.ShapeDtypeStruct((2, rows, 128), jnp.float32),
    )(ea_r, esel_r, etok128)


_sc_mesh = plsc.VectorSubcoreMesh(
    core_axis_name="c", subcore_axis_name="s", num_cores=NC, num_subcores=NS)


@functools.partial(
    pl.kernel,
    out_type=jax.ShapeDtypeStruct((NC, N, D), jnp.float32),
    mesh=_sc_mesh,
    scratch_types=(
        pltpu.VMEM((K1, LANES), jnp.int32),
        pltpu.VMEM((K1, LANES), jnp.int32),
        pltpu.VMEM((CH1, D), jnp.float32),
        pltpu.VMEM_SHARED((NP, D), jnp.float32),
        pltpu.SemaphoreType.DMA,
    ),
)
def _sc_segsum(hflat, srcall, dst2d, z_big, s_out,
               idx_s, idx_d, rows, s_acc, semg):
    c = lax.axis_index("c")
    s = lax.axis_index("s")

    @pl.when(s == 0)
    def _():
        pltpu.sync_copy(z_big, s_acc)

    plsc.subcore_barrier()

    def body(i, carry):
        r0 = (s * ITERS1 + i) * K1
        pltpu.sync_copy(srcall.at[c, pl.ds(r0, K1)], idx_s)
        pltpu.sync_copy(dst2d.at[pl.ds(r0, K1)], idx_d)
        descs = []
        for j in range(K1):
            descs.append(pltpu.async_copy(
                hflat.at[idx_s.at[j]],
                rows.at[pl.ds(j * LANES, LANES)], semg))
        for d_ in descs:
            d_.wait()
        for j in range(K1):
            pltpu.sync_copy(rows.at[pl.ds(j * LANES, LANES)],
                            s_acc.at[idx_d.at[j]], add=True)
        return carry

    lax.fori_loop(0, ITERS1, body, 0)
    plsc.subcore_barrier()

    @pl.when(s < 10)
    def _():
        pltpu.sync_copy(s_acc.at[pl.ds(s * 1000, 1000)],
                        s_out.at[c, pl.ds(s * 1000, 1000)])


def _tc_emb_body(s_ref, se_ref, h_ref, wm_ref, we_ref, ws_ref, out_ref):
    acc = jnp.dot(s_ref[0], wm_ref[...], preferred_element_type=jnp.float32)
    acc += jnp.dot(se_ref[0], we_ref[...], preferred_element_type=jnp.float32)
    acc += jnp.dot(h_ref[0], ws_ref[...], preferred_element_type=jnp.float32)
    out_ref[0] = jax.nn.relu(acc)


def _make_emb(S, Se, H, W_msg, W_edge, W_self):
    blk = 1000
    return pl.pallas_call(
        _tc_emb_body,
        grid=(NC, N // blk),
        in_specs=[
            pl.BlockSpec((1, blk, D), lambda c, i: (c, i, 0)),
            pl.BlockSpec((1, blk, DEP), lambda c, i: (c, i, 0)),
            pl.BlockSpec((1, blk, D), lambda c, i: (c, i, 0)),
            pl.BlockSpec((D, D), lambda c, i: (0, 0)),
            pl.BlockSpec((DEP, D), lambda c, i: (0, 0)),
            pl.BlockSpec((D, D), lambda c, i: (0, 0)),
        ],
        out_specs=pl.BlockSpec((1, blk, D), lambda c, i: (c, i, 0)),
        out_shape=jax.ShapeDtypeStruct((NC, N, D), jnp.float32),
    )(S, Se, H, W_msg, W_edge, W_self)


@functools.partial(
    pl.kernel,
    out_type=(
        jax.ShapeDtypeStruct((PAIR_PAD, D), jnp.float32),
        jax.ShapeDtypeStruct((PAIR_PAD, D), jnp.float32),
    ),
    mesh=_sc_mesh,
    scratch_types=(
        pltpu.VMEM((K2, LANES), jnp.int32),
        pltpu.VMEM((K2, LANES), jnp.int32),
        pltpu.VMEM((CH2, D), jnp.float32),
        pltpu.VMEM((CH2, D), jnp.float32),
        pltpu.SemaphoreType.DMA,
    ),
)
def _sc_pair_gather(embflat, ms2d, md2d, g1_out, g2_out,
                    idx_a, idx_b, rows1, rows2, semg):
    c = lax.axis_index("c")
    s = lax.axis_index("s")
    wid = c * NS + s

    def body(i, carry):
        r0 = (wid * ITERS2 + i) * K2
        pltpu.sync_copy(ms2d.at[pl.ds(r0, K2)], idx_a)
        pltpu.sync_copy(md2d.at[pl.ds(r0, K2)], idx_b)
        descs = []
        for j in range(K2):
            descs.append(pltpu.async_copy(
                embflat.at[idx_a.at[j]],
                rows1.at[pl.ds(j * LANES, LANES)], semg))
            descs.append(pltpu.async_copy(
                embflat.at[idx_b.at[j]],
                rows2.at[pl.ds(j * LANES, LANES)], semg))
        for d_ in descs:
            d_.wait()
        pltpu.sync_copy(rows1, g1_out.at[pl.ds(r0 * LANES, CH2)])
        pltpu.sync_copy(rows2, g2_out.at[pl.ds(r0 * LANES, CH2)])
        return carry

    lax.fori_loop(0, ITERS2, body, 0)


def _tc_node_body(e_ref, oi_ref, w1_ref, b1_ref, w2_ref, b2_ref, wm_ref, out_ref):
    hn = jax.nn.relu(
        jnp.dot(e_ref[...], w1_ref[...], preferred_element_type=jnp.float32)
        + b1_ref[...])
    pred = jnp.dot(hn, w2_ref[...], preferred_element_type=jnp.float32) + b2_ref[...]
    out_ref[0, 0] = jnp.sum(jnp.abs(pred - oi_ref[...]) * wm_ref[...])


def _make_node_loss(emb_nc, orig_inj, W1, b1, W2, b2, wmask):
    return pl.pallas_call(
        _tc_node_body,
        grid=(1,),
        in_specs=[
            pl.BlockSpec((NODE_PAD, D), lambda i: (0, 0)),
            pl.BlockSpec((NODE_PAD, INJ_DIM), lambda i: (0, 0)),
            pl.BlockSpec((D, D), lambda i: (0, 0)),
            pl.BlockSpec((1, D), lambda i: (0, 0)),
            pl.BlockSpec((D, INJ_DIM), lambda i: (0, 0)),
            pl.BlockSpec((1, INJ_DIM), lambda i: (0, 0)),
            pl.BlockSpec((NODE_PAD, 1), lambda i: (0, 0)),
        ],
        out_specs=pl.BlockSpec(memory_space=pltpu.SMEM),
        out_shape=jax.ShapeDtypeStruct((1, 1), jnp.float32),
    )(emb_nc, orig_inj, W1, b1, W2, b2, wmask)


def _tc_edge_body(g1_ref, g2_ref, oe_ref, w1a_ref, w1b_ref, b1_ref,
                  w2_ref, b2_ref, wm_ref, out_ref):
    he = jnp.dot(g1_ref[...], w1a_ref[...], preferred_element_type=jnp.float32)
    he += jnp.dot(g2_ref[...], w1b_ref[...], preferred_element_type=jnp.float32)
    he = jax.nn.relu(he + b1_ref[...])
    pred = jnp.dot(he, w2_ref[...], preferred_element_type=jnp.float32) + b2_ref[...]
    part = jnp.sum(jnp.abs(pred - oe_ref[...]) * wm_ref[...])

    @pl.when(pl.program_id(0) == 0)
    def _():
        out_ref[0, 0] = part

    @pl.when(pl.program_id(0) != 0)
    def _():
        out_ref[0, 0] += part


def _make_edge_loss(G1, G2, orig_ef, W1a, W1b, b1, W2, b2, wmask):
    blk = 4096
    return pl.pallas_call(
        _tc_edge_body,
        grid=(PAIR_PAD // blk,),
        in_specs=[
            pl.BlockSpec((blk, D), lambda i: (i, 0)),
            pl.BlockSpec((blk, D), lambda i: (i, 0)),
            pl.BlockSpec((blk, EDGE_DIM), lambda i: (i, 0)),
            pl.BlockSpec((D, D), lambda i: (0, 0)),
            pl.BlockSpec((D, D), lambda i: (0, 0)),
            pl.BlockSpec((1, D), lambda i: (0, 0)),
            pl.BlockSpec((D, EDGE_DIM), lambda i: (0, 0)),
            pl.BlockSpec((1, EDGE_DIM), lambda i: (0, 0)),
            pl.BlockSpec((blk, 1), lambda i: (i, 0)),
        ],
        out_specs=pl.BlockSpec(memory_space=pltpu.SMEM),
        out_shape=jax.ShapeDtypeStruct((1, 1), jnp.float32),
    )(G1, G2, orig_ef, W1a, W1b, b1, W2, b2, wmask)


def kernel(x, edge_index, edge_attr, W_in, b_in, W_msg, W_edge, W_self,
           nm_W1, nm_b1, nm_W2, nm_b2, node_mask_token,
           em_W1, em_b1, em_W2, em_b2, edge_mask_token):
    src = edge_index[0]
    dst = edge_index[1]

    nm_idx = jax.random.permutation(jax.random.key(123), N)[:NMASK]
    em_idx = jax.random.permutation(jax.random.key(456), E)[:EMASK]
    nm_idx_pad = jnp.concatenate(
        [nm_idx, jnp.zeros((NODE_PAD - NMASK,), nm_idx.dtype)])
    nmask_row = jnp.zeros((N,), jnp.float32).at[nm_idx].set(1.0)[:, None]
    esel = jnp.zeros((E_PAD, DEP), jnp.float32).at[em_idx, :EDGE_DIM].set(1.0)
    esel_r = esel.reshape(E_PAD * DEP // 128, 128)
    colmask = jnp.asarray(_colmask_n)
    tokrow = jnp.concatenate(
        [node_mask_token, jnp.zeros((D - INJ_DIM,), jnp.float32)])[None, :]
    etok = jnp.concatenate(
        [edge_mask_token, jnp.zeros((DEP - EDGE_DIM,), jnp.float32)])
    etok128 = jnp.tile(etok, 128 // DEP)[None, :]

    ea_pad = jnp.pad(edge_attr, ((0, E_PAD - E), (0, DEP - DE)))
    ea_r = ea_pad.reshape(E_PAD * DEP // 128, 128)
    src2d = jnp.pad(src, (0, E_PAD - E)).reshape(E_PAD // LANES, LANES)
    dst2d = jnp.pad(dst, (0, E_PAD - E),
                    constant_values=N).reshape(E_PAD // LANES, LANES)

    H = _make_h(x, nmask_row, tokrow, colmask, W_in, b_in[None, :])
    EA_r = _make_ea(ea_r, esel_r, etok128)
    EA3 = EA_r.reshape(NC, E_PAD, DEP)

    srcall = jnp.stack([src2d, src2d + N])
    S = _sc_segsum(H.reshape(2 * N, D), srcall, dst2d,
                   jnp.zeros((NP, D), jnp.float32))
    dstp = dst2d.reshape(-1)
    Se = jax.vmap(lambda e: jax.ops.segment_sum(e, dstp, num_segments=N))(EA3)

    EMB = _make_emb(S, Se, H, W_msg,
                    jnp.pad(W_edge, ((0, DEP - DE), (0, 0))), W_self)

    ms2d = jnp.pad(jnp.take(src, em_idx),
                   (0, PAIR_PAD - EMASK)).reshape(PAIR_PAD // LANES, LANES)
    md2d = jnp.pad(jnp.take(dst, em_idx),
                   (0, PAIR_PAD - EMASK)).reshape(PAIR_PAD // LANES, LANES)
    G1, G2 = _sc_pair_gather(EMB.reshape(2 * N, D), ms2d + N, md2d + N)

    emb_nc = jnp.take(EMB[0], nm_idx_pad, axis=0)
    orig_inj = jnp.take(x[:, :INJ_DIM], nm_idx_pad, axis=0)
    node_sum = _make_node_loss(emb_nc, orig_inj, nm_W1, nm_b1[None, :],
                               nm_W2, nm_b2[None, :], jnp.asarray(_wmask_node))

    orig_ef = jnp.pad(jnp.take(edge_attr[:, :EDGE_DIM], em_idx, axis=0),
                      ((0, PAIR_PAD - EMASK), (0, 0)))
    edge_sum = _make_edge_loss(G1, G2, orig_ef, em_W1[:D], em_W1[D:],
                               em_b1[None, :], em_W2, em_b2[None, :],
                               jnp.asarray(_wmask_edge))

    node_loss = node_sum[0, 0] / (NMASK * INJ_DIM)
    edge_loss = edge_sum[0, 0] / (EMASK * EDGE_DIM)
    return 0.5 * node_loss + 0.5 * edge_loss
